# Initial kernel scaffold; baseline (speedup 1.0000x reference)
import jax
import jax.numpy as jnp
from jax.experimental import pallas as pl


def kernel(logits, labels):
    raise NotImplementedError("write your pallas kernel here")


# speedup vs baseline: 2.7087x; 2.7087x over previous
import jax
import jax.numpy as jnp
from jax.experimental import pallas as pl

S = 64.0
MEAN = 0.5
SIGMA = 0.05


def _body(lab_ref, ce_ref, se_ref, x_ref, o_ref, *, bc):
    j = pl.program_id(1)
    x = x_ref[...]
    br = x.shape[0]
    lab = lab_ref[0, 0, :]
    cols = jax.lax.broadcasted_iota(jnp.int32, (br, bc), 1) + j * bc
    m = cols == lab[:, None]
    ce = ce_ref[0, 0, :][:, None]
    se = se_ref[0, 0, :][:, None]
    fix = x * ce - jnp.sqrt(jnp.maximum(1.0 - x * x, 0.0)) * se
    o_ref[...] = jnp.where(m, fix, x) * S


def kernel(logits, labels):
    B, C = logits.shape
    BR = 256
    BC = 1024
    grid_r = pl.cdiv(B, BR)
    grid_c = pl.cdiv(C, BC)

    elastic = jax.random.normal(jax.random.key(42), (B,), dtype=logits.dtype)
    elastic = elastic * SIGMA + MEAN
    ce = jnp.cos(elastic).reshape(grid_r, 1, BR)
    se = jnp.sin(elastic).reshape(grid_r, 1, BR)
    labs = labels.astype(jnp.int32).reshape(grid_r, 1, BR)

    import functools
    body = functools.partial(_body, bc=BC)

    return pl.pallas_call(
        body,
        grid=(grid_r, grid_c),
        in_specs=[
            pl.BlockSpec((1, 1, BR), lambda i, j: (i, 0, 0)),
            pl.BlockSpec((1, 1, BR), lambda i, j: (i, 0, 0)),
            pl.BlockSpec((1, 1, BR), lambda i, j: (i, 0, 0)),
            pl.BlockSpec((BR, BC), lambda i, j: (i, j)),
        ],
        out_specs=pl.BlockSpec((BR, BC), lambda i, j: (i, j)),
        out_shape=jax.ShapeDtypeStruct((B, C), logits.dtype),
    )(labs, ce, se, logits)


# speedup vs baseline: 3.0071x; 1.1102x over previous
import jax
import jax.numpy as jnp
from jax.experimental import pallas as pl

S = 64.0
MEAN = 0.5
SIGMA = 0.05


def _body(lab_ref, ce_ref, se_ref, x_ref, o_ref, *, bc):
    j = pl.program_id(1)
    x = x_ref[...]
    br = x.shape[0]
    lab = lab_ref[0, 0, :]
    cols = jax.lax.broadcasted_iota(jnp.int32, (br, bc), 1) + j * bc
    m = cols == lab[:, None]
    ce = ce_ref[0, 0, :][:, None]
    se = se_ref[0, 0, :][:, None]
    fix = x * ce - jnp.sqrt(jnp.maximum(1.0 - x * x, 0.0)) * se
    o_ref[...] = jnp.where(m, fix, x) * S


def kernel(logits, labels):
    B, C = logits.shape
    BR = 256
    BC = 4096
    grid_r = pl.cdiv(B, BR)
    grid_c = pl.cdiv(C, BC)

    elastic = jax.random.normal(jax.random.key(42), (B,), dtype=logits.dtype)
    elastic = elastic * SIGMA + MEAN
    ce = jnp.cos(elastic).reshape(grid_r, 1, BR)
    se = jnp.sin(elastic).reshape(grid_r, 1, BR)
    labs = labels.astype(jnp.int32).reshape(grid_r, 1, BR)

    import functools
    body = functools.partial(_body, bc=BC)

    return pl.pallas_call(
        body,
        grid=(grid_r, grid_c),
        in_specs=[
            pl.BlockSpec((1, 1, BR), lambda i, j: (i, 0, 0)),
            pl.BlockSpec((1, 1, BR), lambda i, j: (i, 0, 0)),
            pl.BlockSpec((1, 1, BR), lambda i, j: (i, 0, 0)),
            pl.BlockSpec((BR, BC), lambda i, j: (i, j)),
        ],
        out_specs=pl.BlockSpec((BR, BC), lambda i, j: (i, j)),
        out_shape=jax.ShapeDtypeStruct((B, C), logits.dtype),
    )(labs, ce, se, logits)
